# initial kernel scaffold (unmeasured)
import jax
import jax.numpy as jnp
from jax import lax
from jax.experimental import pallas as pl
from jax.experimental.pallas import tpu as pltpu

N_DEV = 8
N_TOK = 2048
D = 512
H = 1024
E_LOCAL = 4
CHUNK = N_TOK // N_DEV


def kernel(x, router_W, route_idx, expert_W, shared_W):
    def body(x_ref, rw_ref, idx_ref, ew_ref, sw_ref, out_ref,
             acc_ref, comm_ref, rs_send, rs_recv, ag_send, ag_recv):
        p = lax.axis_index("i")
        left = lax.rem(p - 1 + N_DEV, N_DEV)
        right = lax.rem(p + 1, N_DEV)

        barrier_sem = pltpu.get_barrier_semaphore()
        for nbr in (left, right):
            pl.semaphore_signal(
                barrier_sem, inc=1,
                device_id=(nbr,), device_id_type=pl.DeviceIdType.MESH,
            )
        pl.semaphore_wait(barrier_sem, 2)

        xv = x_ref[:, :]
        scores = jnp.dot(xv, rw_ref[:, :], preferred_element_type=jnp.float32)
        s_max = jnp.max(scores, axis=-1, keepdims=True)
        e_s = jnp.exp(scores - s_max)
        probs = e_s / jnp.sum(e_s, axis=-1, keepdims=True)
        idx = idx_ref[:, :]
        onehot = (lax.broadcasted_iota(jnp.int32, scores.shape, 1) == idx)
        gate = jnp.sum(jnp.where(onehot, probs, 0.0), axis=-1, keepdims=True)

        acc = jnp.zeros((N_TOK, H), jnp.float32)
        for k in range(E_LOCAL):
            e_id = p * E_LOCAL + k
            w_k = jnp.where(idx == e_id, gate, 0.0)
            acc = acc + jnp.dot(w_k * xv, ew_ref[k],
                                preferred_element_type=jnp.float32)
        acc_ref[:, :] = acc

        c0 = lax.rem(p, N_DEV)
        comm_ref[0, :, :] = acc_ref[pl.ds(c0 * CHUNK, CHUNK), :]
        for s in range(N_DEV - 1):
            rdma = pltpu.make_async_remote_copy(
                src_ref=comm_ref.at[s],
                dst_ref=comm_ref.at[s + 1],
                send_sem=rs_send.at[s],
                recv_sem=rs_recv.at[s + 1],
                device_id=(right,),
                device_id_type=pl.DeviceIdType.MESH,
            )
            rdma.start()
            rdma.wait()
            c = lax.rem(p - (s + 1) + N_DEV, N_DEV)
            comm_ref[s + 1, :, :] = (
                comm_ref[s + 1, :, :] + acc_ref[pl.ds(c * CHUNK, CHUNK), :]
            )

        o_p = lax.rem(p + 1, N_DEV)
        x_chunk = x_ref[pl.ds(o_p * CHUNK, CHUNK), :]
        out_ref[pl.ds(o_p * CHUNK, CHUNK), :] = (
            comm_ref[N_DEV - 1, :, :]
            + jnp.dot(x_chunk, sw_ref[:, :], preferred_element_type=jnp.float32)
        )

        for t in range(N_DEV - 1):
            c_send = lax.rem(p - t + 1 + N_DEV, N_DEV)
            rdma = pltpu.make_async_remote_copy(
                src_ref=out_ref.at[pl.ds(c_send * CHUNK, CHUNK), :],
                dst_ref=out_ref.at[pl.ds(c_send * CHUNK, CHUNK), :],
                send_sem=ag_send.at[t],
                recv_sem=ag_recv.at[t],
                device_id=(right,),
                device_id_type=pl.DeviceIdType.MESH,
            )
            rdma.start()
            rdma.wait()

    return pl.pallas_call(
        body,
        out_shape=jax.ShapeDtypeStruct((N_TOK, H), jnp.float32),
        in_specs=[
            pl.BlockSpec(memory_space=pltpu.VMEM),
            pl.BlockSpec(memory_space=pltpu.VMEM),
            pl.BlockSpec(memory_space=pltpu.VMEM),
            pl.BlockSpec(memory_space=pltpu.VMEM),
            pl.BlockSpec(memory_space=pltpu.VMEM),
        ],
        out_specs=pl.BlockSpec(memory_space=pltpu.VMEM),
        scratch_shapes=[
            pltpu.VMEM((N_TOK, H), jnp.float32),
            pltpu.VMEM((N_DEV, CHUNK, H), jnp.float32),
            pltpu.SemaphoreType.DMA((N_DEV,)),
            pltpu.SemaphoreType.DMA((N_DEV,)),
            pltpu.SemaphoreType.DMA((N_DEV - 1,)),
            pltpu.SemaphoreType.DMA((N_DEV - 1,)),
        ],
        compiler_params=pltpu.CompilerParams(collective_id=0),
    )(x, router_W, route_idx, expert_W, shared_W)


# baseline (device time: 218116 ns/iter reference)
import jax
import jax.numpy as jnp
from jax import lax
from jax.experimental import pallas as pl
from jax.experimental.pallas import tpu as pltpu

N_DEV = 8
N_TOK = 2048
D = 512
H = 1024
E_LOCAL = 4
CHUNK = N_TOK // N_DEV


def kernel(x, router_W, route_idx, expert_W, shared_W):
    def body(x_ref, rw_ref, idx_ref, ew_ref, sw_ref, out_ref,
             acc_ref, comm_ref, rs_send, rs_recv, ag_send, ag_recv):
        p = lax.axis_index("i")
        left = lax.rem(p - 1 + N_DEV, N_DEV)
        right = lax.rem(p + 1, N_DEV)

        barrier_sem = pltpu.get_barrier_semaphore()
        for nbr in (left, right):
            pl.semaphore_signal(
                barrier_sem, inc=1,
                device_id=(nbr,), device_id_type=pl.DeviceIdType.MESH,
            )
        pl.semaphore_wait(barrier_sem, 2)

        xv = x_ref[:, :]
        scores = jnp.dot(xv, rw_ref[:, :], preferred_element_type=jnp.float32)
        s_max = jnp.max(scores, axis=-1, keepdims=True)
        e_s = jnp.exp(scores - s_max)
        probs = e_s / jnp.sum(e_s, axis=-1, keepdims=True)
        idx = idx_ref[:, :]
        onehot = (lax.broadcasted_iota(jnp.int32, scores.shape, 1) == idx)
        gate = jnp.sum(jnp.where(onehot, probs, 0.0), axis=-1, keepdims=True)

        acc = jnp.zeros((N_TOK, H), jnp.float32)
        for k in range(E_LOCAL):
            e_id = p * E_LOCAL + k
            w_k = jnp.where(idx == e_id, gate, 0.0)
            acc = acc + jnp.dot(w_k * xv, ew_ref[k],
                                preferred_element_type=jnp.float32)
        acc_ref[:, :] = acc

        c0 = lax.rem(p, N_DEV)
        comm_ref[0, :, :] = acc_ref[pl.ds(c0 * CHUNK, CHUNK), :]
        for s in range(N_DEV - 1):
            rdma = pltpu.make_async_remote_copy(
                src_ref=comm_ref.at[s],
                dst_ref=comm_ref.at[s + 1],
                send_sem=rs_send.at[s],
                recv_sem=rs_recv.at[s + 1],
                device_id=(right,),
                device_id_type=pl.DeviceIdType.MESH,
            )
            rdma.start()
            rdma.wait()
            c = lax.rem(p - (s + 1) + N_DEV, N_DEV)
            comm_ref[s + 1, :, :] = (
                comm_ref[s + 1, :, :] + acc_ref[pl.ds(c * CHUNK, CHUNK), :]
            )

        o_p = lax.rem(p + 1, N_DEV)
        x_chunk = x_ref[pl.ds(o_p * CHUNK, CHUNK), :]
        out_ref[pl.ds(o_p * CHUNK, CHUNK), :] = (
            comm_ref[N_DEV - 1, :, :]
            + jnp.dot(x_chunk, sw_ref[:, :], preferred_element_type=jnp.float32)
        )

        for t in range(N_DEV - 1):
            c_send = lax.rem(p - t + 1 + N_DEV, N_DEV)
            rdma = pltpu.make_async_remote_copy(
                src_ref=out_ref.at[pl.ds(c_send * CHUNK, CHUNK), :],
                dst_ref=out_ref.at[pl.ds(c_send * CHUNK, CHUNK), :],
                send_sem=ag_send.at[t],
                recv_sem=ag_recv.at[t],
                device_id=(right,),
                device_id_type=pl.DeviceIdType.MESH,
            )
            rdma.start()
            rdma.wait()

    return pl.pallas_call(
        body,
        out_shape=jax.ShapeDtypeStruct((N_TOK, H), jnp.float32),
        in_specs=[
            pl.BlockSpec(memory_space=pltpu.VMEM),
            pl.BlockSpec(memory_space=pltpu.VMEM),
            pl.BlockSpec(memory_space=pltpu.VMEM),
            pl.BlockSpec(memory_space=pltpu.VMEM),
            pl.BlockSpec(memory_space=pltpu.VMEM),
        ],
        out_specs=pl.BlockSpec(memory_space=pltpu.VMEM),
        scratch_shapes=[
            pltpu.VMEM((N_TOK, H), jnp.float32),
            pltpu.VMEM((N_DEV, CHUNK, H), jnp.float32),
            pltpu.SemaphoreType.DMA((N_DEV,)),
            pltpu.SemaphoreType.DMA((N_DEV,)),
            pltpu.SemaphoreType.DMA((N_DEV - 1,)),
            pltpu.SemaphoreType.DMA((N_DEV - 1,)),
        ],
        compiler_params=pltpu.CompilerParams(
            collective_id=0,
            vmem_limit_bytes=100 * 1024 * 1024,
        ),
    )(x, router_W, route_idx, expert_W, shared_W)


# device time: 106145 ns/iter; 2.0549x vs baseline; 2.0549x over previous
import jax
import jax.numpy as jnp
from jax import lax
from jax.experimental import pallas as pl
from jax.experimental.pallas import tpu as pltpu

N_DEV = 8
N_TOK = 2048
D = 512
H = 1024
E_LOCAL = 4
CHUNK = N_TOK // N_DEV

ORDERS = ((1, 2, 4), (2, 4, 1), (4, 1, 2))
COLS = ((0, 384), (384, 384), (768, 256))
W_MAX = 384
RS_SLOT_BASE = (0, 4, 6)
AG_SLOT_BASE = (0, 1, 3)


def _subset_masks(masks):
    out = [0]
    for m in masks:
        out = out + [o | m for o in out]
    return out


def _lmap(v):
    return (v & 4) | ((v & 3) ^ ((v & 3) >> 1))


def kernel(x, router_W, route_idx, expert_W, shared_W):
    def body(x_ref, rw_ref, idx_ref, ew_ref, sw_ref, out_ref,
             acc_ref, recv_ref, rs_send, rs_recv, ag_send, ag_recv):
        p = lax.axis_index("i")
        b = _lmap(p)

        barrier_sem = pltpu.get_barrier_semaphore()
        for m in (1, 2, 4):
            pl.semaphore_signal(
                barrier_sem, inc=1,
                device_id=(_lmap(b ^ m),),
                device_id_type=pl.DeviceIdType.MESH,
            )
        pl.semaphore_wait(barrier_sem, 3)

        xv = x_ref[:, :]
        scores = jnp.dot(xv, rw_ref[:, :], preferred_element_type=jnp.float32)
        s_max = jnp.max(scores, axis=-1, keepdims=True)
        e_s = jnp.exp(scores - s_max)
        probs = e_s / jnp.sum(e_s, axis=-1, keepdims=True)
        idx = idx_ref[:, :]
        onehot = (lax.broadcasted_iota(jnp.int32, scores.shape, 1) == idx)
        gate = jnp.sum(jnp.where(onehot, probs, 0.0), axis=-1, keepdims=True)

        acc = jnp.zeros((N_TOK, H), jnp.float32)
        for k in range(E_LOCAL):
            e_id = p * E_LOCAL + k
            w_k = jnp.where(idx == e_id, gate, 0.0)
            acc = acc + jnp.dot(w_k * xv, ew_ref[k],
                                preferred_element_type=jnp.float32)
        acc_ref[:, :] = acc

        for s in range(3):
            handles = []
            for j in range(3):
                m = ORDERS[j][s]
                done = sum(ORDERS[j][:s])
                free = ORDERS[j][s + 1:]
                partner = _lmap(b ^ m)
                c0, w = COLS[j]
                keep = done | m
                for ti, t in enumerate(_subset_masks(free)):
                    slot = RS_SLOT_BASE[s] + ti
                    c_send = ((b ^ m) & keep) | t
                    rdma = pltpu.make_async_remote_copy(
                        src_ref=acc_ref.at[pl.ds(c_send * CHUNK, CHUNK),
                                           pl.ds(c0, w)],
                        dst_ref=recv_ref.at[j, slot, :, pl.ds(0, w)],
                        send_sem=rs_send.at[j, slot],
                        recv_sem=rs_recv.at[j, slot],
                        device_id=(partner,),
                        device_id_type=pl.DeviceIdType.MESH,
                    )
                    rdma.start()
                    c_recv = (b & keep) | t
                    handles.append((rdma, j, slot, c_recv, c0, w))
            for rdma, j, slot, c_recv, c0, w in handles:
                rdma.wait()
                rows = pl.ds(c_recv * CHUNK, CHUNK)
                cols = pl.ds(c0, w)
                acc_ref[rows, cols] = (
                    acc_ref[rows, cols] + recv_ref[j, slot, :, :w]
                )

        rows_b = pl.ds(b * CHUNK, CHUNK)
        out_ref[rows_b, :] = (
            acc_ref[rows_b, :]
            + jnp.dot(x_ref[rows_b, :], sw_ref[:, :],
                      preferred_element_type=jnp.float32)
        )

        for s in range(3):
            handles = []
            for j in range(3):
                rev = ORDERS[j][::-1]
                m = rev[s]
                partner = _lmap(b ^ m)
                c0, w = COLS[j]
                for ti, t in enumerate(_subset_masks(rev[:s])):
                    slot = AG_SLOT_BASE[s] + ti
                    c_send = b ^ t
                    rows = pl.ds(c_send * CHUNK, CHUNK)
                    cols = pl.ds(c0, w)
                    rdma = pltpu.make_async_remote_copy(
                        src_ref=out_ref.at[rows, cols],
                        dst_ref=out_ref.at[rows, cols],
                        send_sem=ag_send.at[j, slot],
                        recv_sem=ag_recv.at[j, slot],
                        device_id=(partner,),
                        device_id_type=pl.DeviceIdType.MESH,
                    )
                    rdma.start()
                    handles.append(rdma)
            for rdma in handles:
                rdma.wait()

    return pl.pallas_call(
        body,
        out_shape=jax.ShapeDtypeStruct((N_TOK, H), jnp.float32),
        in_specs=[
            pl.BlockSpec(memory_space=pltpu.VMEM),
            pl.BlockSpec(memory_space=pltpu.VMEM),
            pl.BlockSpec(memory_space=pltpu.VMEM),
            pl.BlockSpec(memory_space=pltpu.VMEM),
            pl.BlockSpec(memory_space=pltpu.VMEM),
        ],
        out_specs=pl.BlockSpec(memory_space=pltpu.VMEM),
        scratch_shapes=[
            pltpu.VMEM((N_TOK, H), jnp.float32),
            pltpu.VMEM((3, 7, CHUNK, W_MAX), jnp.float32),
            pltpu.SemaphoreType.DMA((3, 7)),
            pltpu.SemaphoreType.DMA((3, 7)),
            pltpu.SemaphoreType.DMA((3, 7)),
            pltpu.SemaphoreType.DMA((3, 7)),
        ],
        compiler_params=pltpu.CompilerParams(
            collective_id=0,
            vmem_limit_bytes=100 * 1024 * 1024,
        ),
    )(x, router_W, route_idx, expert_W, shared_W)


# device time: 77959 ns/iter; 2.7978x vs baseline; 1.3615x over previous
import jax
import jax.numpy as jnp
from jax import lax
from jax.experimental import pallas as pl
from jax.experimental.pallas import tpu as pltpu

N_DEV = 8
N_TOK = 2048
D = 512
H = 1024
E_LOCAL = 4
CHUNK = N_TOK // N_DEV

ORDERS = ((1, 2, 4), (2, 4, 1), (4, 1, 2))
COLS = ((0, 384), (384, 384), (768, 256))
W_MAX = 384
RS_SLOT_BASE = (0, 4, 6)
AG_SLOT_BASE = (0, 1, 3)


def _subset_masks(masks):
    out = [0]
    for m in masks:
        out = out + [o | m for o in out]
    return out


def _lmap(v):
    return (v & 4) | ((v & 3) ^ ((v & 3) >> 1))


def kernel(x, router_W, route_idx, expert_W, shared_W):
    def body(x_ref, rw_ref, idx_ref, ew_ref, sw_ref, out_ref,
             acc_ref, wire_ref, recv_ref, rs_send, rs_recv, ag_send, ag_recv):
        p = lax.axis_index("i")
        b = _lmap(p)

        barrier_sem = pltpu.get_barrier_semaphore()
        for m in (1, 2, 4):
            pl.semaphore_signal(
                barrier_sem, inc=1,
                device_id=(_lmap(b ^ m),),
                device_id_type=pl.DeviceIdType.MESH,
            )
        pl.semaphore_wait(barrier_sem, 3)

        xv = x_ref[:, :]
        scores = jnp.dot(xv, rw_ref[:, :], preferred_element_type=jnp.float32)
        s_max = jnp.max(scores, axis=-1, keepdims=True)
        e_s = jnp.exp(scores - s_max)
        probs = e_s / jnp.sum(e_s, axis=-1, keepdims=True)
        idx = idx_ref[:, :]
        onehot = (lax.broadcasted_iota(jnp.int32, scores.shape, 1) == idx)
        gate = jnp.sum(jnp.where(onehot, probs, 0.0), axis=-1, keepdims=True)

        xb = xv.astype(jnp.bfloat16)
        acc = jnp.zeros((N_TOK, H), jnp.float32)
        for k in range(E_LOCAL):
            e_id = p * E_LOCAL + k
            w_k = jnp.where(idx == e_id, gate, 0.0)
            acc = acc + jnp.dot(w_k.astype(jnp.bfloat16) * xb,
                                ew_ref[k].astype(jnp.bfloat16),
                                preferred_element_type=jnp.float32)
        acc_ref[:, :] = acc

        for s in range(3):
            handles = []
            for j in range(3):
                m = ORDERS[j][s]
                done = sum(ORDERS[j][:s])
                free = ORDERS[j][s + 1:]
                partner = _lmap(b ^ m)
                c0, w = COLS[j]
                keep = done | m
                for ti, t in enumerate(_subset_masks(free)):
                    slot = RS_SLOT_BASE[s] + ti
                    c_send = ((b ^ m) & keep) | t
                    rows_s = pl.ds(c_send * CHUNK, CHUNK)
                    cols_s = pl.ds(c0, w)
                    wire_ref[rows_s, cols_s] = (
                        acc_ref[rows_s, cols_s].astype(jnp.bfloat16)
                    )
                    rdma = pltpu.make_async_remote_copy(
                        src_ref=wire_ref.at[rows_s, cols_s],
                        dst_ref=recv_ref.at[j, slot, :, pl.ds(0, w)],
                        send_sem=rs_send.at[j, slot],
                        recv_sem=rs_recv.at[j, slot],
                        device_id=(partner,),
                        device_id_type=pl.DeviceIdType.MESH,
                    )
                    rdma.start()
                    c_recv = (b & keep) | t
                    handles.append((rdma, j, slot, c_recv, c0, w))
            for rdma, j, slot, c_recv, c0, w in handles:
                rdma.wait()
                rows = pl.ds(c_recv * CHUNK, CHUNK)
                cols = pl.ds(c0, w)
                acc_ref[rows, cols] = (
                    acc_ref[rows, cols]
                    + recv_ref[j, slot, :, :w].astype(jnp.float32)
                )

        rows_b = pl.ds(b * CHUNK, CHUNK)
        wire_ref[rows_b, :] = (
            acc_ref[rows_b, :]
            + jnp.dot(x_ref[rows_b, :], sw_ref[:, :],
                      preferred_element_type=jnp.float32)
        ).astype(jnp.bfloat16)

        for s in range(3):
            handles = []
            for j in range(3):
                rev = ORDERS[j][::-1]
                m = rev[s]
                partner = _lmap(b ^ m)
                c0, w = COLS[j]
                for ti, t in enumerate(_subset_masks(rev[:s])):
                    slot = AG_SLOT_BASE[s] + ti
                    c_send = b ^ t
                    rows = pl.ds(c_send * CHUNK, CHUNK)
                    cols = pl.ds(c0, w)
                    rdma = pltpu.make_async_remote_copy(
                        src_ref=wire_ref.at[rows, cols],
                        dst_ref=wire_ref.at[rows, cols],
                        send_sem=ag_send.at[j, slot],
                        recv_sem=ag_recv.at[j, slot],
                        device_id=(partner,),
                        device_id_type=pl.DeviceIdType.MESH,
                    )
                    rdma.start()
                    handles.append(rdma)
            for rdma in handles:
                rdma.wait()

        out_ref[:, :] = wire_ref[:, :].astype(jnp.float32)

    return pl.pallas_call(
        body,
        out_shape=jax.ShapeDtypeStruct((N_TOK, H), jnp.float32),
        in_specs=[
            pl.BlockSpec(memory_space=pltpu.VMEM),
            pl.BlockSpec(memory_space=pltpu.VMEM),
            pl.BlockSpec(memory_space=pltpu.VMEM),
            pl.BlockSpec(memory_space=pltpu.VMEM),
            pl.BlockSpec(memory_space=pltpu.VMEM),
        ],
        out_specs=pl.BlockSpec(memory_space=pltpu.VMEM),
        scratch_shapes=[
            pltpu.VMEM((N_TOK, H), jnp.float32),
            pltpu.VMEM((N_TOK, H), jnp.bfloat16),
            pltpu.VMEM((3, 7, CHUNK, W_MAX), jnp.bfloat16),
            pltpu.SemaphoreType.DMA((3, 7)),
            pltpu.SemaphoreType.DMA((3, 7)),
            pltpu.SemaphoreType.DMA((3, 7)),
            pltpu.SemaphoreType.DMA((3, 7)),
        ],
        compiler_params=pltpu.CompilerParams(
            collective_id=0,
            vmem_limit_bytes=100 * 1024 * 1024,
        ),
    )(x, router_W, route_idx, expert_W, shared_W)
